# baseline (speedup 1.0000x reference)
import functools

import jax
import jax.numpy as jnp
from jax import lax
from jax.experimental import pallas as pl
from jax.experimental.pallas import tpu as pltpu
from jax.experimental.pallas import tpu_sc as plsc

_N = 67108864
_NC = 2
_CSHARE = _N // _NC
_SCHUNK = 524288
_NITER = _CSHARE // _SCHUNK


def _sc_body(x_hbm, o_hbm, buf0, buf1, head, isem, osem):
    cid = lax.axis_index("c")
    sid = lax.axis_index("s")
    base = cid * _CSHARE

    def in_copy(i, buf, sem):
        return pltpu.make_async_copy(
            x_hbm.at[pl.ds(base + i * _SCHUNK, _SCHUNK)], buf, sem)

    def out_copy(i, buf, sem):
        return pltpu.make_async_copy(
            buf, o_hbm.at[pl.ds(base + i * _SCHUNK, _SCHUNK)], sem)

    @pl.when(sid == 0)
    def _stream():
        in_copy(0, buf0, isem.at[0]).start()
        in_copy(1, buf1, isem.at[1]).start()
        in_copy(0, buf0, isem.at[0]).wait()
        out_copy(0, buf0, osem.at[0]).start()
        in_copy(1, buf1, isem.at[1]).wait()
        out_copy(1, buf1, osem.at[1]).start()

        def body(j, carry):
            i0, i1 = 2 * j, 2 * j + 1
            out_copy(i0 - 2, buf0, osem.at[0]).wait()
            in_copy(i0, buf0, isem.at[0]).start()
            out_copy(i1 - 2, buf1, osem.at[1]).wait()
            in_copy(i1, buf1, isem.at[1]).start()
            in_copy(i0, buf0, isem.at[0]).wait()
            out_copy(i0, buf0, osem.at[0]).start()
            in_copy(i1, buf1, isem.at[1]).wait()
            out_copy(i1, buf1, osem.at[1]).start()
            return carry

        lax.fori_loop(1, _NITER // 2, body, 0)
        out_copy(_NITER - 2, buf0, osem.at[0]).wait()
        out_copy(_NITER - 1, buf1, osem.at[1]).wait()

    @pl.when((cid == 0) & (sid == 0))
    def _patch():
        pltpu.sync_copy(x_hbm.at[pl.ds(0, 16)], head)
        idx = lax.iota(jnp.int32, 16)
        head[...] = jnp.where(idx == 0, jnp.float32(0.0), head[...])
        pltpu.sync_copy(head, o_hbm.at[pl.ds(0, 16)])


_sc_copy = functools.partial(
    pl.kernel,
    out_type=jax.ShapeDtypeStruct((_N,), jnp.float32),
    mesh=plsc.VectorSubcoreMesh(core_axis_name="c", subcore_axis_name="s"),
    scratch_types=[
        pltpu.VMEM_SHARED((_SCHUNK,), jnp.float32),
        pltpu.VMEM_SHARED((_SCHUNK,), jnp.float32),
        pltpu.VMEM((16,), jnp.float32),
        pltpu.SemaphoreType.DMA((2,)),
        pltpu.SemaphoreType.DMA((2,)),
    ],
)(_sc_body)


def kernel(x):
    return _sc_copy(x)


# speedup vs baseline: 1.5388x; 1.5388x over previous
import jax
import jax.numpy as jnp
from jax.experimental import pallas as pl
from jax.experimental.pallas import tpu as pltpu

_N = 67108864
_NBUF = 3
_BUF = 4 * 1024 * 1024
_K = 512 * 1024
_SIZES = ([_K, _K, 2 * _K, 4 * _K] + [8 * _K] * 14 + [4 * _K, 2 * _K, _K, _K])
_OFFS = [sum(_SIZES[:i]) for i in range(len(_SIZES))]
_NCHUNK = len(_SIZES)
_DEPTH = 2


def _copy_kernel(x_hbm, o_hbm, *scratch):
    bufs, isem, osem = scratch[:_NBUF], scratch[_NBUF], scratch[_NBUF + 1]

    def in_copy(i):
        b = i % _NBUF
        return pltpu.make_async_copy(
            x_hbm.at[pl.ds(_OFFS[i], _SIZES[i])],
            bufs[b].at[pl.ds(0, _SIZES[i])], isem.at[b])

    def out_copy(i):
        b = i % _NBUF
        return pltpu.make_async_copy(
            bufs[b].at[pl.ds(0, _SIZES[i])],
            o_hbm.at[pl.ds(_OFFS[i], _SIZES[i])], osem.at[b])

    for j in range(_DEPTH):
        in_copy(j).start()
    for i in range(_NCHUNK):
        j = i + _DEPTH
        if j < _NCHUNK:
            if j >= _NBUF:
                out_copy(j - _NBUF).wait()
            in_copy(j).start()
        in_copy(i).wait()
        if i == 0:
            buf = bufs[0]
            idx = jax.lax.broadcasted_iota(jnp.int32, (1024,), 0)
            buf[0:1024] = jnp.where(idx == 0, jnp.float32(0.0), buf[0:1024])
        out_copy(i).start()
    for i in range(_NCHUNK - _NBUF, _NCHUNK):
        out_copy(i).wait()


def kernel(x):
    return pl.pallas_call(
        _copy_kernel,
        in_specs=[pl.BlockSpec(memory_space=pl.ANY)],
        out_specs=pl.BlockSpec(memory_space=pl.ANY),
        out_shape=jax.ShapeDtypeStruct((_N,), x.dtype),
        scratch_shapes=(
            [pltpu.VMEM((_BUF,), jnp.float32) for _ in range(_NBUF)]
            + [pltpu.SemaphoreType.DMA((_NBUF,)),
               pltpu.SemaphoreType.DMA((_NBUF,))]
        ),
    )(x)


# speedup vs baseline: 1.5561x; 1.0112x over previous
import jax
import jax.numpy as jnp
from jax.experimental import pallas as pl
from jax.experimental.pallas import tpu as pltpu

_N = 67108864
_NBUF = 3
_CHUNK = 4 * 1024 * 1024
_NCHUNK = _N // _CHUNK
_DEPTH = 1


def _copy_kernel(x_hbm, o_hbm, *scratch):
    bufs, isem, osem = scratch[:_NBUF], scratch[_NBUF], scratch[_NBUF + 1]

    def in_copy(i):
        return pltpu.make_async_copy(
            x_hbm.at[pl.ds(i * _CHUNK, _CHUNK)], bufs[i % _NBUF],
            isem.at[i % _NBUF])

    def out_copy(i):
        return pltpu.make_async_copy(
            bufs[i % _NBUF], o_hbm.at[pl.ds(i * _CHUNK, _CHUNK)],
            osem.at[i % _NBUF])

    for j in range(_DEPTH):
        in_copy(j).start()
    for i in range(_NCHUNK):
        j = i + _DEPTH
        if j < _NCHUNK:
            if j >= _NBUF:
                out_copy(j - _NBUF).wait()
            in_copy(j).start()
        in_copy(i).wait()
        if i == 0:
            buf = bufs[0]
            idx = jax.lax.broadcasted_iota(jnp.int32, (1024,), 0)
            buf[0:1024] = jnp.where(idx == 0, jnp.float32(0.0), buf[0:1024])
        out_copy(i).start()
    for i in range(_NCHUNK - _NBUF, _NCHUNK):
        out_copy(i).wait()


def kernel(x):
    return pl.pallas_call(
        _copy_kernel,
        in_specs=[pl.BlockSpec(memory_space=pl.ANY)],
        out_specs=pl.BlockSpec(memory_space=pl.ANY),
        out_shape=jax.ShapeDtypeStruct((_N,), x.dtype),
        scratch_shapes=(
            [pltpu.VMEM((_CHUNK,), jnp.float32) for _ in range(_NBUF)]
            + [pltpu.SemaphoreType.DMA((_NBUF,)),
               pltpu.SemaphoreType.DMA((_NBUF,))]
        ),
    )(x)


# speedup vs baseline: 1.5631x; 1.0045x over previous
import jax
import jax.numpy as jnp
from jax.experimental import pallas as pl
from jax.experimental.pallas import tpu as pltpu

_N = 67108864
_NBUF = 3
_CHUNK = 4 * 1024 * 1024
_NCHUNK = _N // _CHUNK
_DEPTH = 2


def _copy_kernel(x_hbm, o_hbm, *scratch):
    bufs, isem, osem = scratch[:_NBUF], scratch[_NBUF], scratch[_NBUF + 1]

    def in_copy(i):
        return pltpu.make_async_copy(
            x_hbm.at[pl.ds(i * _CHUNK, _CHUNK)], bufs[i % _NBUF],
            isem.at[i % _NBUF])

    def out_copy(i):
        return pltpu.make_async_copy(
            bufs[i % _NBUF], o_hbm.at[pl.ds(i * _CHUNK, _CHUNK)],
            osem.at[i % _NBUF])

    for j in range(_DEPTH):
        in_copy(j).start()
    for i in range(_NCHUNK):
        j = i + _DEPTH
        if j < _NCHUNK:
            if j >= _NBUF:
                out_copy(j - _NBUF).wait()
            in_copy(j).start()
        in_copy(i).wait()
        if i == 0:
            buf = bufs[0]
            idx = jax.lax.broadcasted_iota(jnp.int32, (1024,), 0)
            buf[0:1024] = jnp.where(idx == 0, jnp.float32(0.0), buf[0:1024])
        out_copy(i).start()
    for i in range(_NCHUNK - _NBUF, _NCHUNK):
        out_copy(i).wait()


def kernel(x):
    return pl.pallas_call(
        _copy_kernel,
        in_specs=[pl.BlockSpec(memory_space=pl.ANY)],
        out_specs=pl.BlockSpec(memory_space=pl.ANY),
        out_shape=jax.ShapeDtypeStruct((_N,), x.dtype),
        scratch_shapes=(
            [pltpu.VMEM((_CHUNK,), jnp.float32) for _ in range(_NBUF)]
            + [pltpu.SemaphoreType.DMA((_NBUF,)),
               pltpu.SemaphoreType.DMA((_NBUF,))]
        ),
    )(x)
